# baseline (speedup 1.0000x reference)
import jax
import jax.numpy as jnp
from jax.experimental import pallas as pl
from jax.experimental.pallas import tpu as pltpu

_CHUNKS = 8


def _sage_body(
    x_ref, adj_ref, wa_ref, wb_ref, bs_ref, g_ref,
    adjb_ref, inv_ref, h0_ref, n0_ref, h1_ref, n1_ref,
):
    n = adj_ref.shape[1]
    f = wa_ref.shape[2]
    c = n // _CHUNKS
    h0_ref[...] = x_ref[0].astype(jnp.bfloat16)
    rhs0 = jnp.concatenate(
        [h0_ref[...], jnp.ones((n, f), jnp.bfloat16)], axis=1
    )
    for k in range(_CHUNKS):
        rows = pl.ds(k * c, c)
        adjb_ref[rows, :] = adj_ref[0, k * c : (k + 1) * c, :].astype(jnp.bfloat16)
        r0 = jnp.dot(adjb_ref[rows, :], rhs0, preferred_element_type=jnp.float32)
        iv = 1.0 / (r0[:, f:] + 1.0)
        inv_ref[rows, :] = iv
        n0_ref[rows, :] = (r0[:, :f] * iv).astype(jnp.bfloat16)

    inv = inv_ref[...]
    hA, nA, hB, nB = h0_ref, n0_ref, h1_ref, n1_ref
    for i in (0, 1):
        for k in range(_CHUNKS):
            rows = pl.ds(k * c, c)
            z = (
                jnp.dot(hA[rows, :], wa_ref[i], preferred_element_type=jnp.float32)
                + jnp.dot(nA[rows, :], wb_ref[i], preferred_element_type=jnp.float32)
                + bs_ref[i]
            )
            hB[rows, :] = jnp.maximum(z, 0.0).astype(jnp.bfloat16)
        hb = hB[...]
        for k in range(_CHUNKS):
            rows = pl.ds(k * c, c)
            nr = jnp.dot(adjb_ref[rows, :], hb, preferred_element_type=jnp.float32)
            nB[rows, :] = (nr * inv[k * c : (k + 1) * c, :]).astype(jnp.bfloat16)
        hA, nA, hB, nB = hB, nB, hA, nA
    m = None
    for k in range(_CHUNKS):
        rows = pl.ds(k * c, c)
        z = (
            jnp.dot(hA[rows, :], wa_ref[2], preferred_element_type=jnp.float32)
            + jnp.dot(nA[rows, :], wb_ref[2], preferred_element_type=jnp.float32)
            + bs_ref[2]
        )
        h = jnp.maximum(z, 0.0)
        hm = jnp.max(h, axis=0, keepdims=True)
        m = hm if m is None else jnp.maximum(m, hm)
    g_ref[0] = m


def _head_body(g_ref, wh_ref, bh_ref, wo_ref, bo_ref, o_ref):
    t = jnp.dot(g_ref[...], wh_ref[...], preferred_element_type=jnp.float32)
    t = t + bh_ref[...]
    o = jnp.dot(t, wo_ref[...], preferred_element_type=jnp.float32)
    o_ref[...] = o + bo_ref[...]


def kernel(x, adj, W0, b0, W1, b1, W2, b2, Wh, bh, Wo, bo):
    B, N, D = x.shape
    F = W0.shape[1]
    Wa = jnp.stack([W0[:D], W1[:F], W2[:F]]).astype(jnp.bfloat16)
    Wb = jnp.stack([W0[D:], W1[F:], W2[F:]]).astype(jnp.bfloat16)
    bs = jnp.stack([b0, b1, b2]).reshape(3, 1, F)

    g = pl.pallas_call(
        _sage_body,
        grid=(B,),
        in_specs=[
            pl.BlockSpec((1, N, D), lambda b: (b, 0, 0)),
            pl.BlockSpec((1, N, N), lambda b: (b, 0, 0)),
            pl.BlockSpec((3, F, F), lambda b: (0, 0, 0)),
            pl.BlockSpec((3, F, F), lambda b: (0, 0, 0)),
            pl.BlockSpec((3, 1, F), lambda b: (0, 0, 0)),
        ],
        out_specs=pl.BlockSpec((1, 1, F), lambda b: (b, 0, 0)),
        out_shape=jax.ShapeDtypeStruct((B, 1, F), jnp.float32),
        scratch_shapes=[
            pltpu.VMEM((N, N), jnp.bfloat16),
            pltpu.VMEM((N, F), jnp.float32),
            pltpu.VMEM((N, F), jnp.bfloat16),
            pltpu.VMEM((N, F), jnp.bfloat16),
            pltpu.VMEM((N, F), jnp.bfloat16),
            pltpu.VMEM((N, F), jnp.bfloat16),
        ],
        compiler_params=pltpu.CompilerParams(
            dimension_semantics=("parallel",),
            vmem_limit_bytes=60 * 1024 * 1024,
        ),
    )(x, adj, Wa, Wb, bs)

    H = Wh.shape[1]
    O = Wo.shape[1]
    out = pl.pallas_call(
        _head_body,
        in_specs=[
            pl.BlockSpec((B, F), lambda: (0, 0)),
            pl.BlockSpec((F, H), lambda: (0, 0)),
            pl.BlockSpec((1, H), lambda: (0, 0)),
            pl.BlockSpec((H, O), lambda: (0, 0)),
            pl.BlockSpec((1, O), lambda: (0, 0)),
        ],
        out_specs=pl.BlockSpec((B, O), lambda: (0, 0)),
        out_shape=jax.ShapeDtypeStruct((B, O), jnp.float32),
    )(g.reshape(B, F), Wh, bh.reshape(1, -1), Wo, bo.reshape(1, -1))
    return out


# speedup vs baseline: 1.0105x; 1.0105x over previous
import jax
import jax.numpy as jnp
from jax.experimental import pallas as pl
from jax.experimental.pallas import tpu as pltpu

_CHUNKS = 8


def _sage_body(
    x_ref, adj_ref, wa_ref, wb_ref, bs_ref, g_ref,
    inv_ref, h0_ref, n0_ref, h1_ref, n1_ref,
):
    n = adj_ref.shape[1]
    f = wa_ref.shape[2]
    c = n // _CHUNKS
    h0_ref[...] = x_ref[0].astype(jnp.bfloat16)
    rhs0 = jnp.concatenate(
        [x_ref[0], jnp.ones((n, f), jnp.float32)], axis=1
    )
    for k in range(_CHUNKS):
        rows = pl.ds(k * c, c)
        r0 = jnp.dot(adj_ref[0, k * c : (k + 1) * c, :], rhs0, preferred_element_type=jnp.float32)
        iv = 1.0 / (r0[:, f:] + 1.0)
        inv_ref[rows, :] = iv
        n0_ref[rows, :] = (r0[:, :f] * iv).astype(jnp.bfloat16)

    inv = inv_ref[...]
    hA, nA, hB, nB = h0_ref, n0_ref, h1_ref, n1_ref
    for i in (0, 1):
        for k in range(_CHUNKS):
            rows = pl.ds(k * c, c)
            z = (
                jnp.dot(hA[rows, :], wa_ref[i], preferred_element_type=jnp.float32)
                + jnp.dot(nA[rows, :], wb_ref[i], preferred_element_type=jnp.float32)
                + bs_ref[i]
            )
            hB[rows, :] = jnp.maximum(z, 0.0).astype(jnp.bfloat16)
        hb = hB[...]
        for k in range(_CHUNKS):
            rows = pl.ds(k * c, c)
            nr = jnp.dot(adj_ref[0, k * c : (k + 1) * c, :], hb.astype(jnp.float32), preferred_element_type=jnp.float32)
            nB[rows, :] = (nr * inv[k * c : (k + 1) * c, :]).astype(jnp.bfloat16)
        hA, nA, hB, nB = hB, nB, hA, nA
    m = None
    for k in range(_CHUNKS):
        rows = pl.ds(k * c, c)
        z = (
            jnp.dot(hA[rows, :], wa_ref[2], preferred_element_type=jnp.float32)
            + jnp.dot(nA[rows, :], wb_ref[2], preferred_element_type=jnp.float32)
            + bs_ref[2]
        )
        h = jnp.maximum(z, 0.0)
        hm = jnp.max(h, axis=0, keepdims=True)
        m = hm if m is None else jnp.maximum(m, hm)
    g_ref[0] = m


def _head_body(g_ref, wh_ref, bh_ref, wo_ref, bo_ref, o_ref):
    t = jnp.dot(g_ref[...], wh_ref[...], preferred_element_type=jnp.float32)
    t = t + bh_ref[...]
    o = jnp.dot(t, wo_ref[...], preferred_element_type=jnp.float32)
    o_ref[...] = o + bo_ref[...]


def kernel(x, adj, W0, b0, W1, b1, W2, b2, Wh, bh, Wo, bo):
    B, N, D = x.shape
    F = W0.shape[1]
    Wa = jnp.stack([W0[:D], W1[:F], W2[:F]]).astype(jnp.bfloat16)
    Wb = jnp.stack([W0[D:], W1[F:], W2[F:]]).astype(jnp.bfloat16)
    bs = jnp.stack([b0, b1, b2]).reshape(3, 1, F)

    g = pl.pallas_call(
        _sage_body,
        grid=(B,),
        in_specs=[
            pl.BlockSpec((1, N, D), lambda b: (b, 0, 0)),
            pl.BlockSpec((1, N, N), lambda b: (b, 0, 0)),
            pl.BlockSpec((3, F, F), lambda b: (0, 0, 0)),
            pl.BlockSpec((3, F, F), lambda b: (0, 0, 0)),
            pl.BlockSpec((3, 1, F), lambda b: (0, 0, 0)),
        ],
        out_specs=pl.BlockSpec((1, 1, F), lambda b: (b, 0, 0)),
        out_shape=jax.ShapeDtypeStruct((B, 1, F), jnp.float32),
        scratch_shapes=[
            pltpu.VMEM((N, F), jnp.float32),
            pltpu.VMEM((N, F), jnp.bfloat16),
            pltpu.VMEM((N, F), jnp.bfloat16),
            pltpu.VMEM((N, F), jnp.bfloat16),
            pltpu.VMEM((N, F), jnp.bfloat16),
        ],
        compiler_params=pltpu.CompilerParams(
            dimension_semantics=("parallel",),
            vmem_limit_bytes=60 * 1024 * 1024,
        ),
    )(x, adj, Wa, Wb, bs)

    H = Wh.shape[1]
    O = Wo.shape[1]
    out = pl.pallas_call(
        _head_body,
        in_specs=[
            pl.BlockSpec((B, F), lambda: (0, 0)),
            pl.BlockSpec((F, H), lambda: (0, 0)),
            pl.BlockSpec((1, H), lambda: (0, 0)),
            pl.BlockSpec((H, O), lambda: (0, 0)),
            pl.BlockSpec((1, O), lambda: (0, 0)),
        ],
        out_specs=pl.BlockSpec((B, O), lambda: (0, 0)),
        out_shape=jax.ShapeDtypeStruct((B, O), jnp.float32),
    )(g.reshape(B, F), Wh, bh.reshape(1, -1), Wo, bo.reshape(1, -1))
    return out


# speedup vs baseline: 1.1035x; 1.0920x over previous
import jax
import jax.numpy as jnp
from jax.experimental import pallas as pl
from jax.experimental.pallas import tpu as pltpu

_CHUNKS = 8


def _sage_body(
    x_ref, adj_ref, w0_ref, b0_ref, w1_ref, b1_ref, w2_ref, b2_ref, g_ref,
    inv_ref, h0_ref, n0_ref, h1_ref, n1_ref,
):
    n = adj_ref.shape[1]
    f = w0_ref.shape[1]
    c = n // _CHUNKS
    w_refs = (w0_ref, w1_ref, w2_ref)
    b_refs = (b0_ref, b1_ref, b2_ref)
    rhs0 = jnp.concatenate([x_ref[0], jnp.ones((n, f), jnp.float32)], axis=1)
    for k in range(_CHUNKS):
        rows = pl.ds(k * c, c)
        r0 = jnp.dot(
            adj_ref[0, k * c : (k + 1) * c, :], rhs0,
            preferred_element_type=jnp.float32,
        )
        iv = 1.0 / (r0[:, f:] + 1.0)
        inv_ref[rows, :] = iv
        n0_ref[rows, :] = r0[:, :f] * iv

    inv = inv_ref[...]
    h0_ref[...] = x_ref[0]
    hA, nA, hB, nB = h0_ref, n0_ref, h1_ref, n1_ref
    for i in (0, 1):
        for k in range(_CHUNKS):
            rows = pl.ds(k * c, c)
            z = (
                jnp.dot(hA[rows, :], w_refs[i][:f, :], preferred_element_type=jnp.float32)
                + jnp.dot(nA[rows, :], w_refs[i][f:, :], preferred_element_type=jnp.float32)
                + b_refs[i][...]
            )
            hB[rows, :] = jnp.maximum(z, 0.0)
        hb = hB[...]
        for k in range(_CHUNKS):
            rows = pl.ds(k * c, c)
            nr = jnp.dot(
                adj_ref[0, k * c : (k + 1) * c, :], hb,
                preferred_element_type=jnp.float32,
            )
            nB[rows, :] = nr * inv[k * c : (k + 1) * c, :]
        hA, nA, hB, nB = hB, nB, hA, nA
    m = None
    for k in range(_CHUNKS):
        rows = pl.ds(k * c, c)
        z = (
            jnp.dot(hA[rows, :], w2_ref[:f, :], preferred_element_type=jnp.float32)
            + jnp.dot(nA[rows, :], w2_ref[f:, :], preferred_element_type=jnp.float32)
            + b2_ref[...]
        )
        h = jnp.maximum(z, 0.0)
        hm = jnp.max(h, axis=0, keepdims=True)
        m = hm if m is None else jnp.maximum(m, hm)
    g_ref[0] = m


def _head_body(g_ref, wh_ref, bh_ref, wo_ref, bo_ref, o_ref):
    t = jnp.dot(g_ref[:, 0, :], wh_ref[...], preferred_element_type=jnp.float32)
    t = t + bh_ref[...]
    o = jnp.dot(t, wo_ref[...], preferred_element_type=jnp.float32)
    o_ref[...] = o + bo_ref[...]


def kernel(x, adj, W0, b0, W1, b1, W2, b2, Wh, bh, Wo, bo):
    B, N, D = x.shape
    F = W0.shape[1]
    g = pl.pallas_call(
        _sage_body,
        grid=(B,),
        in_specs=[
            pl.BlockSpec((1, N, D), lambda b: (b, 0, 0)),
            pl.BlockSpec((1, N, N), lambda b: (b, 0, 0)),
            pl.BlockSpec((2 * D, F), lambda b: (0, 0)),
            pl.BlockSpec((F,), lambda b: (0,)),
            pl.BlockSpec((2 * F, F), lambda b: (0, 0)),
            pl.BlockSpec((F,), lambda b: (0,)),
            pl.BlockSpec((2 * F, F), lambda b: (0, 0)),
            pl.BlockSpec((F,), lambda b: (0,)),
        ],
        out_specs=pl.BlockSpec((1, 1, F), lambda b: (b, 0, 0)),
        out_shape=jax.ShapeDtypeStruct((B, 1, F), jnp.float32),
        scratch_shapes=[
            pltpu.VMEM((N, F), jnp.float32),
            pltpu.VMEM((N, F), jnp.float32),
            pltpu.VMEM((N, F), jnp.float32),
            pltpu.VMEM((N, F), jnp.float32),
            pltpu.VMEM((N, F), jnp.float32),
        ],
        compiler_params=pltpu.CompilerParams(
            dimension_semantics=("parallel",),
            vmem_limit_bytes=60 * 1024 * 1024,
        ),
    )(x, adj, W0, b0, W1, b1, W2, b2)

    H = Wh.shape[1]
    O = Wo.shape[1]
    out = pl.pallas_call(
        _head_body,
        in_specs=[
            pl.BlockSpec((B, 1, F), lambda: (0, 0, 0)),
            pl.BlockSpec((F, H), lambda: (0, 0)),
            pl.BlockSpec((H,), lambda: (0,)),
            pl.BlockSpec((H, O), lambda: (0, 0)),
            pl.BlockSpec((O,), lambda: (0,)),
        ],
        out_specs=pl.BlockSpec((B, O), lambda: (0, 0)),
        out_shape=jax.ShapeDtypeStruct((B, O), jnp.float32),
    )(g, Wh, bh, Wo, bo)
    return out
